# Initial kernel scaffold; baseline (speedup 1.0000x reference)
import jax
import jax.numpy as jnp
from jax.experimental import pallas as pl


def kernel(x, edge_index, W):
    raise NotImplementedError("write your pallas kernel here")


# speedup vs baseline: 3.6703x; 3.6703x over previous
import functools

import jax
import jax.numpy as jnp
from jax import lax
from jax.experimental import pallas as pl
from jax.experimental.pallas import tpu as pltpu
from jax.experimental.pallas import tpu_sc as plsc

_ALPHA = 0.2
_L = 16
_NC = 2
_NS = 16
_B = 80
_D = 128
_DR = 80
_NP = 10368
_DR0 = 10240


def _matmul(x, w):
    n, d_in = x.shape
    d_out = w.shape[1]
    rows = n // 10

    def body(x_ref, w_ref, o_ref):
        o_ref[...] = jnp.dot(x_ref[...], w_ref[...],
                             preferred_element_type=jnp.float32)

    return pl.pallas_call(
        body,
        grid=(10,),
        in_specs=[
            pl.BlockSpec((rows, d_in), lambda i: (i, 0)),
            pl.BlockSpec((d_in, d_out), lambda i: (0, 0)),
        ],
        out_specs=pl.BlockSpec((rows, d_out), lambda i: (i, 0)),
        out_shape=jax.ShapeDtypeStruct((n, d_out), jnp.float32),
    )(x, w)


def _permute(v, idx):
    return lax.gather(
        v, idx[:, None],
        lax.GatherDimensionNumbers(offset_dims=(), collapsed_slice_dims=(0,),
                                   start_index_map=(0,)),
        (1,), mode=lax.GatherScatterMode.PROMISE_IN_BOUNDS)


def _sc_attention(h, src3, dst3):
    n = h.shape[0]
    nw, nblk, _ = src3.shape
    nck = _D // _L
    zpt = _NP // _NS

    mesh = plsc.VectorSubcoreMesh(core_axis_name="c", subcore_axis_name="s")

    @functools.partial(
        pl.kernel,
        out_type=jax.ShapeDtypeStruct((_NC * _NP, _D), jnp.float32),
        mesh=mesh,
        scratch_types=[
            pltpu.VMEM((_B,), jnp.int32),
            pltpu.VMEM((_B,), jnp.int32),
            pltpu.VMEM((_B, _D), jnp.float32),
            pltpu.VMEM((_B, _D), jnp.float32),
            pltpu.VMEM((_DR * _D,), jnp.float32),
            pltpu.VMEM((_DR,), jnp.int32),
            pltpu.VMEM_SHARED((_NP, _D), jnp.float32),
            pltpu.SemaphoreType.DMA,
            pltpu.SemaphoreType.DMA,
        ],
    )
    def k(h_hbm, src_hbm, dst_hbm, acc_out,
          src_w, dst_w, rows_s, rows_d, den_f, den_idx, acc_sh, sem_s, sem_t):
        cid = lax.axis_index("c")
        sid = lax.axis_index("s")
        wid = sid * _NC + cid
        lane = lax.iota(jnp.int32, _L)
        zf = jnp.zeros((_L,), jnp.float32)

        def zrow(i, carry):
            for kq in range(nck):
                rows_s[i, pl.ds(kq * _L, _L)] = zf
            return carry
        lax.fori_loop(0, _B, zrow, 0)

        def zden(i, carry):
            den_f[pl.ds(i * _L, _L)] = zf
            return carry
        lax.fori_loop(0, _DR * _D // _L, zden, 0)

        for kq in range(_DR // _L):
            den_idx[pl.ds(kq * _L, _L)] = _DR0 + kq * _L + lane

        zb = sid * zpt
        for part in range(zpt // _B):
            pltpu.sync_copy(rows_s, acc_sh.at[pl.ds(zb + part * _B, _B)])
        pltpu.sync_copy(rows_s.at[pl.ds(0, zpt % _B)],
                        acc_sh.at[pl.ds(zb + zpt - zpt % _B, zpt % _B)])
        plsc.subcore_barrier()

        def block(blk, carry):
            pltpu.sync_copy(src_hbm.at[wid, blk], src_w)
            pltpu.sync_copy(dst_hbm.at[wid, blk], dst_w)
            cp_s = pltpu.async_copy(h_hbm.at[src_w], rows_s, sem_s)
            cp_t = pltpu.async_copy(h_hbm.at[dst_w], rows_d, sem_t)
            cp_s.wait()
            cp_t.wait()

            def group(g, gcarry):
                base = g * _L
                sv16 = src_w[pl.ds(base, _L)]
                for j in range(_L):
                    ei = base + j
                    tv = []
                    acc = zf
                    for kq in range(nck):
                        sv = rows_s[ei, pl.ds(kq * _L, _L)]
                        t = rows_d[ei, pl.ds(kq * _L, _L)]
                        tv.append(t)
                        df = sv - t
                        acc = acc + df * df
                    for shf in (1, 2, 4, 8):
                        acc = acc + _permute(acc, lane ^ shf)
                    ee = jnp.exp(_ALPHA * acc)
                    for kq in range(nck):
                        rows_d[ei, pl.ds(kq * _L, _L)] = ee * tv[kq]
                    srcs = sv16[j]
                    p16 = (srcs >> 4) << 4
                    sel = lane == (srcs & 15)
                    den_f[pl.ds(p16, _L)] = (den_f[pl.ds(p16, _L)]
                                             + jnp.where(sel, ee, 0.0))
                return gcarry
            lax.fori_loop(0, _B // _L, group, 0)

            pltpu.sync_copy(rows_d, acc_sh.at[src_w], add=True)
            return carry
        lax.fori_loop(0, nblk, block, 0)

        def cvt(i, carry):
            for kq in range(nck):
                rows_s[i, pl.ds(kq * _L, _L)] = den_f[pl.ds(i * _D + kq * _L,
                                                            _L)]
            return carry
        lax.fori_loop(0, _DR, cvt, 0)
        pltpu.sync_copy(rows_s, acc_sh.at[den_idx], add=True)

        plsc.subcore_barrier()
        pltpu.sync_copy(acc_sh.at[pl.ds(zb, zpt)],
                        acc_out.at[pl.ds(cid * _NP + zb, zpt)])

    return k(h, src3, dst3)


def _combine(acc, n):
    def body(acc_ref, o_ref):
        s = acc_ref[0] + acc_ref[1]
        num = s[:n]
        dsum = s[_DR0:_DR0 + _DR]
        row = lax.broadcasted_iota(jnp.int32, (n, _DR), 0)
        blkid = lax.broadcasted_iota(jnp.int32, (n, _DR), 1)
        a = ((row >> 7) == blkid).astype(jnp.float32)
        dblk = jnp.dot(a, dsum, preferred_element_type=jnp.float32)
        r2 = lax.broadcasted_iota(jnp.int32, (n, _D), 0)
        c2 = lax.broadcasted_iota(jnp.int32, (n, _D), 1)
        cmask = ((r2 & 127) == c2).astype(jnp.float32)
        dcol = jnp.sum(dblk * cmask, axis=1, keepdims=True)
        o_ref[...] = jnp.maximum(num / dcol, 0.0)

    return pl.pallas_call(
        body,
        out_shape=jax.ShapeDtypeStruct((n, _D), jnp.float32),
    )(acc)


def kernel(x, edge_index, W):
    n = x.shape[0]
    e = edge_index.shape[1]
    nw = _NC * _NS
    epw = e // nw
    h = _matmul(x, W)
    src3 = edge_index[0].reshape(nw, epw // _B, _B)
    dst3 = edge_index[1].reshape(nw, epw // _B, _B)
    acc = _sc_attention(h, src3, dst3)
    return _combine(acc.reshape(_NC, _NP, _D), n)


# speedup vs baseline: 4.5573x; 1.2417x over previous
import functools

import jax
import jax.numpy as jnp
from jax import lax
from jax.experimental import pallas as pl
from jax.experimental.pallas import tpu as pltpu
from jax.experimental.pallas import tpu_sc as plsc

_ALPHA = 0.2
_L = 16
_NC = 2
_NS = 16
_B = 80
_D = 128
_DR = 80
_NP = 10368
_DR0 = 10240


def _matmul(x, w):
    n, d_in = x.shape
    d_out = w.shape[1]
    rows = n // 10

    def body(x_ref, w_ref, o_ref):
        o_ref[...] = jnp.dot(x_ref[...], w_ref[...],
                             preferred_element_type=jnp.float32)

    return pl.pallas_call(
        body,
        grid=(10,),
        in_specs=[
            pl.BlockSpec((rows, d_in), lambda i: (i, 0)),
            pl.BlockSpec((d_in, d_out), lambda i: (0, 0)),
        ],
        out_specs=pl.BlockSpec((rows, d_out), lambda i: (i, 0)),
        out_shape=jax.ShapeDtypeStruct((n, d_out), jnp.float32),
    )(x, w)


def _permute(v, idx):
    return lax.gather(
        v, idx[:, None],
        lax.GatherDimensionNumbers(offset_dims=(), collapsed_slice_dims=(0,),
                                   start_index_map=(0,)),
        (1,), mode=lax.GatherScatterMode.PROMISE_IN_BOUNDS)


def _sc_attention(h, src3, dst3):
    n = h.shape[0]
    nw, nblk, _ = src3.shape
    nck = _D // _L
    zpt = _NP // _NS

    mesh = plsc.VectorSubcoreMesh(core_axis_name="c", subcore_axis_name="s")

    @functools.partial(
        pl.kernel,
        out_type=jax.ShapeDtypeStruct((_NC * _NP, _D), jnp.float32),
        mesh=mesh,
        scratch_types=[
            pltpu.VMEM((_B,), jnp.int32),
            pltpu.VMEM((_B,), jnp.int32),
            pltpu.VMEM((_B,), jnp.int32),
            pltpu.VMEM((_B,), jnp.int32),
            pltpu.VMEM((_B,), jnp.int32),
            pltpu.VMEM((_B, _D), jnp.float32),
            pltpu.VMEM((_B, _D), jnp.float32),
            pltpu.VMEM((_B, _D), jnp.float32),
            pltpu.VMEM((_DR * _D,), jnp.float32),
            pltpu.VMEM((_DR,), jnp.int32),
            pltpu.VMEM_SHARED((_NP, _D), jnp.float32),
            pltpu.SemaphoreType.DMA,
            pltpu.SemaphoreType.DMA,
            pltpu.SemaphoreType.DMA,
            pltpu.SemaphoreType.DMA,
        ],
    )
    def k(h_hbm, src_hbm, dst_hbm, acc_out,
          sid0, sid1, did0, did1, sc_idx, rows_s, rows_d0, rows_d1, den_f,
          den_idx, acc_sh, sem_gs, sem_gd, sem_sc, sem_id):
        cid = lax.axis_index("c")
        sid = lax.axis_index("s")
        wid = sid * _NC + cid
        lane = lax.iota(jnp.int32, _L)
        zf = jnp.zeros((_L,), jnp.float32)

        def zrow(i, carry):
            for kq in range(nck):
                rows_s[i, pl.ds(kq * _L, _L)] = zf
            return carry
        lax.fori_loop(0, _B, zrow, 0)

        def zden(i, carry):
            den_f[pl.ds(i * _L, _L)] = zf
            return carry
        lax.fori_loop(0, _DR * _D // _L, zden, 0)

        for kq in range(_DR // _L):
            den_idx[pl.ds(kq * _L, _L)] = _DR0 + kq * _L + lane

        zb = sid * zpt
        for part in range(zpt // _B):
            pltpu.sync_copy(rows_s, acc_sh.at[pl.ds(zb + part * _B, _B)])
        pltpu.sync_copy(rows_s.at[pl.ds(0, zpt % _B)],
                        acc_sh.at[pl.ds(zb + zpt - zpt % _B, zpt % _B)])
        plsc.subcore_barrier()

        def compute_block(sidb, cur):
            def group(g, gcarry):
                base = g * _L
                sv16 = sidb[pl.ds(base, _L)]
                sc_idx[pl.ds(base, _L)] = sv16
                for j in range(_L):
                    ei = base + j
                    tv = []
                    acc0 = zf
                    acc1 = zf
                    for kq in range(nck):
                        sv = rows_s[ei, pl.ds(kq * _L, _L)]
                        t = cur[ei, pl.ds(kq * _L, _L)]
                        tv.append(t)
                        df = sv - t
                        if kq % 2 == 0:
                            acc0 = acc0 + df * df
                        else:
                            acc1 = acc1 + df * df
                    acc = acc0 + acc1
                    for shf in (1, 2, 4, 8):
                        acc = acc + _permute(acc, lane ^ shf)
                    ee = jnp.exp(_ALPHA * acc)
                    for kq in range(nck):
                        cur[ei, pl.ds(kq * _L, _L)] = ee * tv[kq]
                    srcs = sv16[j]
                    p16 = (srcs >> 4) << 4
                    sel = lane == (srcs & 15)
                    den_f[pl.ds(p16, _L)] = (den_f[pl.ds(p16, _L)]
                                             + jnp.where(sel, ee, 0.0))
                return gcarry
            lax.fori_loop(0, _B // _L, group, 0)

        def body(b, cur, nxt, sidb, didb, sidn, didn, first):
            pltpu.make_async_copy(h_hbm.at[sidb], rows_s, sem_gs).wait()
            pltpu.make_async_copy(h_hbm.at[didb], cur, sem_gd).wait()

            if not first:
                pltpu.make_async_copy(nxt, acc_sh.at[sc_idx], sem_sc).wait()

            @pl.when(b < nblk - 1)
            def _issue_next_dst_gather():
                pltpu.make_async_copy(src_hbm.at[wid, b + 1], sidn,
                                      sem_id).wait()
                pltpu.make_async_copy(dst_hbm.at[wid, b + 1], didn,
                                      sem_id).wait()
                pltpu.async_copy(h_hbm.at[didn], nxt, sem_gd)

            compute_block(sidb, cur)
            pltpu.async_copy(cur, acc_sh.at[sc_idx], sem_sc, add=True)

            @pl.when(b < nblk - 2)
            def _prefetch_ids():
                pltpu.async_copy(src_hbm.at[wid, b + 2], sidb, sem_id)
                pltpu.async_copy(dst_hbm.at[wid, b + 2], didb, sem_id)

            @pl.when(b < nblk - 1)
            def _issue_next_src_gather():
                pltpu.async_copy(h_hbm.at[sidn], rows_s, sem_gs)

        pltpu.sync_copy(src_hbm.at[wid, 0], sid0)
        pltpu.sync_copy(dst_hbm.at[wid, 0], did0)
        pltpu.async_copy(h_hbm.at[sid0], rows_s, sem_gs)
        pltpu.async_copy(h_hbm.at[did0], rows_d0, sem_gd)
        pltpu.async_copy(src_hbm.at[wid, 1], sid1, sem_id)
        pltpu.async_copy(dst_hbm.at[wid, 1], did1, sem_id)
        body(jnp.int32(0), rows_d0, rows_d1, sid0, did0, sid1, did1, True)

        def pair(kk, carry):
            b1 = 2 * kk + 1
            body(b1, rows_d1, rows_d0, sid1, did1, sid0, did0, False)
            body(b1 + 1, rows_d0, rows_d1, sid0, did0, sid1, did1, False)
            return carry
        lax.fori_loop(0, (nblk - 1) // 2, pair, 0)

        pltpu.make_async_copy(rows_d0, acc_sh.at[sc_idx], sem_sc).wait()

        def cvt(i, carry):
            for kq in range(nck):
                rows_s[i, pl.ds(kq * _L, _L)] = den_f[pl.ds(i * _D + kq * _L,
                                                            _L)]
            return carry
        lax.fori_loop(0, _DR, cvt, 0)
        pltpu.sync_copy(rows_s, acc_sh.at[den_idx], add=True)

        plsc.subcore_barrier()
        pltpu.sync_copy(acc_sh.at[pl.ds(zb, zpt)],
                        acc_out.at[pl.ds(cid * _NP + zb, zpt)])

    return k(h, src3, dst3)


def _combine(acc, n):
    def body(acc_ref, o_ref):
        s = acc_ref[0] + acc_ref[1]
        num = s[:n]
        dsum = s[_DR0:_DR0 + _DR]
        row = lax.broadcasted_iota(jnp.int32, (n, _DR), 0)
        blkid = lax.broadcasted_iota(jnp.int32, (n, _DR), 1)
        a = ((row >> 7) == blkid).astype(jnp.float32)
        dblk = jnp.dot(a, dsum, preferred_element_type=jnp.float32)
        r2 = lax.broadcasted_iota(jnp.int32, (n, _D), 0)
        c2 = lax.broadcasted_iota(jnp.int32, (n, _D), 1)
        cmask = ((r2 & 127) == c2).astype(jnp.float32)
        dcol = jnp.sum(dblk * cmask, axis=1, keepdims=True)
        o_ref[...] = jnp.maximum(num / dcol, 0.0)

    return pl.pallas_call(
        body,
        out_shape=jax.ShapeDtypeStruct((n, _D), jnp.float32),
    )(acc)


def kernel(x, edge_index, W):
    n = x.shape[0]
    e = edge_index.shape[1]
    nw = _NC * _NS
    epw = e // nw
    h = _matmul(x, W)
    src3 = edge_index[0].reshape(nw, epw // _B, _B)
    dst3 = edge_index[1].reshape(nw, epw // _B, _B)
    acc = _sc_attention(h, src3, dst3)
    return _combine(acc.reshape(_NC, _NP, _D), n)


# speedup vs baseline: 5.8696x; 1.2880x over previous
import functools

import jax
import jax.numpy as jnp
from jax import lax
from jax.experimental import pallas as pl
from jax.experimental.pallas import tpu as pltpu
from jax.experimental.pallas import tpu_sc as plsc

_ALPHA = 0.2
_L = 16
_NC = 2
_NS = 16
_B = 80
_D = 128
_DR = 80
_NP = 10368
_DR0 = 10240


def _matmul(x, w):
    n, d_in = x.shape
    d_out = w.shape[1]
    rows = n // 10

    def body(x_ref, w_ref, o_ref):
        o_ref[...] = jnp.dot(x_ref[...], w_ref[...],
                             preferred_element_type=jnp.float32)

    return pl.pallas_call(
        body,
        grid=(10,),
        in_specs=[
            pl.BlockSpec((rows, d_in), lambda i: (i, 0)),
            pl.BlockSpec((d_in, d_out), lambda i: (0, 0)),
        ],
        out_specs=pl.BlockSpec((rows, d_out), lambda i: (i, 0)),
        out_shape=jax.ShapeDtypeStruct((n, d_out), jnp.float32),
    )(x, w)


def _permute(v, idx):
    return lax.gather(
        v, idx[:, None],
        lax.GatherDimensionNumbers(offset_dims=(), collapsed_slice_dims=(0,),
                                   start_index_map=(0,)),
        (1,), mode=lax.GatherScatterMode.PROMISE_IN_BOUNDS)


def _sc_attention(h, src3, dst3):
    n = h.shape[0]
    nw, nblk, _ = src3.shape
    nck = _D // _L
    zpt = _NP // _NS

    mesh = plsc.VectorSubcoreMesh(core_axis_name="c", subcore_axis_name="s")

    @functools.partial(
        pl.kernel,
        out_type=jax.ShapeDtypeStruct((_NC * _NP, _D), jnp.float32),
        mesh=mesh,
        scratch_types=[
            pltpu.VMEM((_B,), jnp.int32),
            pltpu.VMEM((_B,), jnp.int32),
            pltpu.VMEM((_B,), jnp.int32),
            pltpu.VMEM((_B,), jnp.int32),
            pltpu.VMEM((_B,), jnp.int32),
            pltpu.VMEM((_B, _D), jnp.float32),
            pltpu.VMEM((_B, _D), jnp.float32),
            pltpu.VMEM((_B, _D), jnp.float32),
            pltpu.VMEM((_DR * _D,), jnp.float32),
            pltpu.VMEM((_DR,), jnp.int32),
            pltpu.VMEM_SHARED((_NP, _D), jnp.float32),
            pltpu.SemaphoreType.DMA,
            pltpu.SemaphoreType.DMA,
            pltpu.SemaphoreType.DMA,
            pltpu.SemaphoreType.DMA,
        ],
    )
    def k(h_hbm, src_hbm, dst_hbm, acc_out,
          sid0, sid1, did0, did1, sc_idx, rows_s, rows_d0, rows_d1, den_f,
          den_idx, acc_sh, sem_gs, sem_gd, sem_sc, sem_id):
        cid = lax.axis_index("c")
        sid = lax.axis_index("s")
        wid = sid * _NC + cid
        lane = lax.iota(jnp.int32, _L)
        zf = jnp.zeros((_L,), jnp.float32)

        def zrow(i, carry):
            for kq in range(nck):
                rows_s[i, pl.ds(kq * _L, _L)] = zf
            return carry
        lax.fori_loop(0, _B, zrow, 0)

        def zden(i, carry):
            den_f[pl.ds(i * _L, _L)] = zf
            return carry
        lax.fori_loop(0, _DR * _D // _L, zden, 0)

        for kq in range(_DR // _L):
            den_idx[pl.ds(kq * _L, _L)] = _DR0 + kq * _L + lane

        zb = sid * zpt
        for part in range(zpt // _B):
            pltpu.sync_copy(rows_s, acc_sh.at[pl.ds(zb + part * _B, _B)])
        pltpu.sync_copy(rows_s.at[pl.ds(0, zpt % _B)],
                        acc_sh.at[pl.ds(zb + zpt - zpt % _B, zpt % _B)])
        plsc.subcore_barrier()

        def compute_block(sidb, cur):
            def group(g, gcarry):
                base = g * _L
                sv16 = sidb[pl.ds(base, _L)]
                sc_idx[pl.ds(base, _L)] = sv16
                accs = []
                for j in range(_L):
                    ei = base + j
                    acc = zf
                    for kq in range(nck):
                        sv = rows_s[ei, pl.ds(kq * _L, _L)]
                        t = cur[ei, pl.ds(kq * _L, _L)]
                        df = sv - t
                        acc = acc + df * df
                    accs.append(acc)
                bit = 1
                while len(accs) > 1:
                    cond = (lane & bit) != 0
                    folded = []
                    for i in range(0, len(accs), 2):
                        a, b2 = accs[i], accs[i + 1]
                        x = jnp.where(cond, b2, a)
                        z = jnp.where(cond, a, b2)
                        folded.append(x + _permute(z, lane ^ bit))
                    accs = folded
                    bit <<= 1
                ee16 = jnp.exp(_ALPHA * accs[0])
                for j in range(_L):
                    ei = base + j
                    eb = _permute(ee16, jnp.full((_L,), j, jnp.int32))
                    for kq in range(nck):
                        cur[ei, pl.ds(kq * _L, _L)] = (
                            eb * cur[ei, pl.ds(kq * _L, _L)])
                    srcs = sv16[j]
                    p16 = (srcs >> 4) << 4
                    sel = lane == (srcs & 15)
                    den_f[pl.ds(p16, _L)] = (den_f[pl.ds(p16, _L)]
                                             + jnp.where(sel, eb, 0.0))
                return gcarry
            lax.fori_loop(0, _B // _L, group, 0)

        def body(b, cur, nxt, sidb, didb, sidn, didn, first):
            pltpu.make_async_copy(h_hbm.at[sidb], rows_s, sem_gs).wait()
            pltpu.make_async_copy(h_hbm.at[didb], cur, sem_gd).wait()

            if not first:
                pltpu.make_async_copy(nxt, acc_sh.at[sc_idx], sem_sc).wait()

            @pl.when(b < nblk - 1)
            def _issue_next_dst_gather():
                pltpu.make_async_copy(src_hbm.at[wid, b + 1], sidn,
                                      sem_id).wait()
                pltpu.make_async_copy(dst_hbm.at[wid, b + 1], didn,
                                      sem_id).wait()
                pltpu.async_copy(h_hbm.at[didn], nxt, sem_gd)

            compute_block(sidb, cur)
            pltpu.async_copy(cur, acc_sh.at[sc_idx], sem_sc, add=True)

            @pl.when(b < nblk - 2)
            def _prefetch_ids():
                pltpu.async_copy(src_hbm.at[wid, b + 2], sidb, sem_id)
                pltpu.async_copy(dst_hbm.at[wid, b + 2], didb, sem_id)

            @pl.when(b < nblk - 1)
            def _issue_next_src_gather():
                pltpu.async_copy(h_hbm.at[sidn], rows_s, sem_gs)

        pltpu.sync_copy(src_hbm.at[wid, 0], sid0)
        pltpu.sync_copy(dst_hbm.at[wid, 0], did0)
        pltpu.async_copy(h_hbm.at[sid0], rows_s, sem_gs)
        pltpu.async_copy(h_hbm.at[did0], rows_d0, sem_gd)
        pltpu.async_copy(src_hbm.at[wid, 1], sid1, sem_id)
        pltpu.async_copy(dst_hbm.at[wid, 1], did1, sem_id)
        body(jnp.int32(0), rows_d0, rows_d1, sid0, did0, sid1, did1, True)

        def pair(kk, carry):
            b1 = 2 * kk + 1
            body(b1, rows_d1, rows_d0, sid1, did1, sid0, did0, False)
            body(b1 + 1, rows_d0, rows_d1, sid0, did0, sid1, did1, False)
            return carry
        lax.fori_loop(0, (nblk - 1) // 2, pair, 0)

        pltpu.make_async_copy(rows_d0, acc_sh.at[sc_idx], sem_sc).wait()

        def cvt(i, carry):
            for kq in range(nck):
                rows_s[i, pl.ds(kq * _L, _L)] = den_f[pl.ds(i * _D + kq * _L,
                                                            _L)]
            return carry
        lax.fori_loop(0, _DR, cvt, 0)
        pltpu.sync_copy(rows_s, acc_sh.at[den_idx], add=True)

        plsc.subcore_barrier()
        pltpu.sync_copy(acc_sh.at[pl.ds(zb, zpt)],
                        acc_out.at[pl.ds(cid * _NP + zb, zpt)])

    return k(h, src3, dst3)


def _combine(acc, n):
    def body(acc_ref, o_ref):
        s = acc_ref[0] + acc_ref[1]
        num = s[:n]
        dsum = s[_DR0:_DR0 + _DR]
        row = lax.broadcasted_iota(jnp.int32, (n, _DR), 0)
        blkid = lax.broadcasted_iota(jnp.int32, (n, _DR), 1)
        a = ((row >> 7) == blkid).astype(jnp.float32)
        dblk = jnp.dot(a, dsum, preferred_element_type=jnp.float32)
        r2 = lax.broadcasted_iota(jnp.int32, (n, _D), 0)
        c2 = lax.broadcasted_iota(jnp.int32, (n, _D), 1)
        cmask = ((r2 & 127) == c2).astype(jnp.float32)
        dcol = jnp.sum(dblk * cmask, axis=1, keepdims=True)
        o_ref[...] = jnp.maximum(num / dcol, 0.0)

    return pl.pallas_call(
        body,
        out_shape=jax.ShapeDtypeStruct((n, _D), jnp.float32),
    )(acc)


def kernel(x, edge_index, W):
    n = x.shape[0]
    e = edge_index.shape[1]
    nw = _NC * _NS
    epw = e // nw
    h = _matmul(x, W)
    src3 = edge_index[0].reshape(nw, epw // _B, _B)
    dst3 = edge_index[1].reshape(nw, epw // _B, _B)
    acc = _sc_attention(h, src3, dst3)
    return _combine(acc.reshape(_NC, _NP, _D), n)


# speedup vs baseline: 6.1579x; 1.0491x over previous
import functools

import jax
import jax.numpy as jnp
from jax import lax
from jax.experimental import pallas as pl
from jax.experimental.pallas import tpu as pltpu
from jax.experimental.pallas import tpu_sc as plsc

_ALPHA = 0.2
_L = 16
_NC = 2
_NS = 16
_B = 80
_D = 128
_DR = 80
_NP = 10368
_DR0 = 10240


def _matmul(x, w):
    n, d_in = x.shape
    d_out = w.shape[1]
    rows = n // 10

    def body(x_ref, w_ref, o_ref):
        o_ref[...] = jnp.dot(x_ref[...], w_ref[...],
                             preferred_element_type=jnp.float32)

    return pl.pallas_call(
        body,
        grid=(10,),
        in_specs=[
            pl.BlockSpec((rows, d_in), lambda i: (i, 0)),
            pl.BlockSpec((d_in, d_out), lambda i: (0, 0)),
        ],
        out_specs=pl.BlockSpec((rows, d_out), lambda i: (i, 0)),
        out_shape=jax.ShapeDtypeStruct((n, d_out), jnp.float32),
    )(x, w)


def _permute(v, idx):
    return lax.gather(
        v, idx[:, None],
        lax.GatherDimensionNumbers(offset_dims=(), collapsed_slice_dims=(0,),
                                   start_index_map=(0,)),
        (1,), mode=lax.GatherScatterMode.PROMISE_IN_BOUNDS)


def _sc_attention(h, src3, dst3):
    n = h.shape[0]
    nw, nblk, _ = src3.shape
    nck = _D // _L
    zpt = _NP // _NS

    mesh = plsc.VectorSubcoreMesh(core_axis_name="c", subcore_axis_name="s")

    @functools.partial(
        pl.kernel,
        out_type=jax.ShapeDtypeStruct((_NC * _NP, _D), jnp.float32),
        mesh=mesh,
        scratch_types=[
            pltpu.VMEM((_B,), jnp.int32),
            pltpu.VMEM((_B,), jnp.int32),
            pltpu.VMEM((_B,), jnp.int32),
            pltpu.VMEM((_B,), jnp.int32),
            pltpu.VMEM((_B,), jnp.int32),
            pltpu.VMEM((_B, _D), jnp.float32),
            pltpu.VMEM((_B, _D), jnp.float32),
            pltpu.VMEM((_B, _D), jnp.float32),
            pltpu.VMEM((_DR * _D,), jnp.float32),
            pltpu.VMEM((_DR,), jnp.int32),
            pltpu.VMEM_SHARED((_NP, _D), jnp.float32),
            pltpu.SemaphoreType.DMA,
            pltpu.SemaphoreType.DMA,
            pltpu.SemaphoreType.DMA,
            pltpu.SemaphoreType.DMA,
        ],
    )
    def k(h_hbm, src_hbm, dst_hbm, acc_out,
          sid0, sid1, did0, did1, sc_idx, rows_s, rows_d0, rows_d1, den_f,
          den_idx, acc_sh, sem_gs, sem_gd, sem_sc, sem_id):
        cid = lax.axis_index("c")
        sid = lax.axis_index("s")
        wid = sid * _NC + cid
        lane = lax.iota(jnp.int32, _L)
        zf = jnp.zeros((_L,), jnp.float32)

        def zrow(i, carry):
            for kq in range(nck):
                rows_s[i, pl.ds(kq * _L, _L)] = zf
            return carry
        lax.fori_loop(0, _B, zrow, 0)

        def zden(i, carry):
            den_f[pl.ds(i * _L, _L)] = zf
            return carry
        lax.fori_loop(0, _DR * _D // _L, zden, 0)

        for kq in range(_DR // _L):
            den_idx[pl.ds(kq * _L, _L)] = _DR0 + kq * _L + lane

        zb = sid * zpt
        for part in range(zpt // _B):
            pltpu.sync_copy(rows_s, acc_sh.at[pl.ds(zb + part * _B, _B)])
        pltpu.sync_copy(rows_s.at[pl.ds(0, zpt % _B)],
                        acc_sh.at[pl.ds(zb + zpt - zpt % _B, zpt % _B)])
        plsc.subcore_barrier()

        def compute_block(sidb, cur):
            def group(g, gcarry):
                base = g * _L
                sv16 = sidb[pl.ds(base, _L)]
                sc_idx[pl.ds(base, _L)] = sv16
                accs = [zf] * _L
                for kq in range(nck):
                    for j in range(_L):
                        ei = base + j
                        sv = rows_s[ei, pl.ds(kq * _L, _L)]
                        t = cur[ei, pl.ds(kq * _L, _L)]
                        df = sv - t
                        accs[j] = accs[j] + df * df
                bit = 1
                while len(accs) > 1:
                    cond = (lane & bit) != 0
                    folded = []
                    for i in range(0, len(accs), 2):
                        a, b2 = accs[i], accs[i + 1]
                        x = jnp.where(cond, b2, a)
                        z = jnp.where(cond, a, b2)
                        folded.append(x + _permute(z, lane ^ bit))
                    accs = folded
                    bit <<= 1
                ee16 = jnp.exp(_ALPHA * accs[0])
                ebs = [_permute(ee16, jnp.full((_L,), j, jnp.int32))
                       for j in range(_L)]
                for j in range(_L):
                    srcs = sv16[j]
                    p16 = (srcs >> 4) << 4
                    sel = lane == (srcs & 15)
                    den_f[pl.ds(p16, _L)] = (den_f[pl.ds(p16, _L)]
                                             + jnp.where(sel, ebs[j], 0.0))
                for kq in range(nck):
                    for j in range(_L):
                        ei = base + j
                        cur[ei, pl.ds(kq * _L, _L)] = (
                            ebs[j] * cur[ei, pl.ds(kq * _L, _L)])
                return gcarry
            lax.fori_loop(0, _B // _L, group, 0)

        def body(b, cur, nxt, sidb, didb, sidn, didn, first):
            pltpu.make_async_copy(h_hbm.at[sidb], rows_s, sem_gs).wait()
            pltpu.make_async_copy(h_hbm.at[didb], cur, sem_gd).wait()

            if not first:
                pltpu.make_async_copy(nxt, acc_sh.at[sc_idx], sem_sc).wait()

            @pl.when(b < nblk - 1)
            def _issue_next_dst_gather():
                pltpu.make_async_copy(src_hbm.at[wid, b + 1], sidn,
                                      sem_id).wait()
                pltpu.make_async_copy(dst_hbm.at[wid, b + 1], didn,
                                      sem_id).wait()
                pltpu.async_copy(h_hbm.at[didn], nxt, sem_gd)

            compute_block(sidb, cur)
            pltpu.async_copy(cur, acc_sh.at[sc_idx], sem_sc, add=True)

            @pl.when(b < nblk - 2)
            def _prefetch_ids():
                pltpu.async_copy(src_hbm.at[wid, b + 2], sidb, sem_id)
                pltpu.async_copy(dst_hbm.at[wid, b + 2], didb, sem_id)

            @pl.when(b < nblk - 1)
            def _issue_next_src_gather():
                pltpu.async_copy(h_hbm.at[sidn], rows_s, sem_gs)

        pltpu.sync_copy(src_hbm.at[wid, 0], sid0)
        pltpu.sync_copy(dst_hbm.at[wid, 0], did0)
        pltpu.async_copy(h_hbm.at[sid0], rows_s, sem_gs)
        pltpu.async_copy(h_hbm.at[did0], rows_d0, sem_gd)
        pltpu.async_copy(src_hbm.at[wid, 1], sid1, sem_id)
        pltpu.async_copy(dst_hbm.at[wid, 1], did1, sem_id)
        body(jnp.int32(0), rows_d0, rows_d1, sid0, did0, sid1, did1, True)

        def pair(kk, carry):
            b1 = 2 * kk + 1
            body(b1, rows_d1, rows_d0, sid1, did1, sid0, did0, False)
            body(b1 + 1, rows_d0, rows_d1, sid0, did0, sid1, did1, False)
            return carry
        lax.fori_loop(0, (nblk - 1) // 2, pair, 0)

        pltpu.make_async_copy(rows_d0, acc_sh.at[sc_idx], sem_sc).wait()

        def cvt(i, carry):
            for kq in range(nck):
                rows_s[i, pl.ds(kq * _L, _L)] = den_f[pl.ds(i * _D + kq * _L,
                                                            _L)]
            return carry
        lax.fori_loop(0, _DR, cvt, 0)
        pltpu.sync_copy(rows_s, acc_sh.at[den_idx], add=True)

        plsc.subcore_barrier()
        pltpu.sync_copy(acc_sh.at[pl.ds(zb, zpt)],
                        acc_out.at[pl.ds(cid * _NP + zb, zpt)])

    return k(h, src3, dst3)


def _combine(acc, n):
    def body(acc_ref, o_ref):
        s = acc_ref[0] + acc_ref[1]
        num = s[:n]
        dsum = s[_DR0:_DR0 + _DR]
        row = lax.broadcasted_iota(jnp.int32, (n, _DR), 0)
        blkid = lax.broadcasted_iota(jnp.int32, (n, _DR), 1)
        a = ((row >> 7) == blkid).astype(jnp.float32)
        dblk = jnp.dot(a, dsum, preferred_element_type=jnp.float32)
        r2 = lax.broadcasted_iota(jnp.int32, (n, _D), 0)
        c2 = lax.broadcasted_iota(jnp.int32, (n, _D), 1)
        cmask = ((r2 & 127) == c2).astype(jnp.float32)
        dcol = jnp.sum(dblk * cmask, axis=1, keepdims=True)
        o_ref[...] = jnp.maximum(num / dcol, 0.0)

    return pl.pallas_call(
        body,
        out_shape=jax.ShapeDtypeStruct((n, _D), jnp.float32),
    )(acc)


def kernel(x, edge_index, W):
    n = x.shape[0]
    e = edge_index.shape[1]
    nw = _NC * _NS
    epw = e // nw
    h = _matmul(x, W)
    src3 = edge_index[0].reshape(nw, epw // _B, _B)
    dst3 = edge_index[1].reshape(nw, epw // _B, _B)
    acc = _sc_attention(h, src3, dst3)
    return _combine(acc.reshape(_NC, _NP, _D), n)


# speedup vs baseline: 7.2525x; 1.1777x over previous
import functools

import jax
import jax.numpy as jnp
from jax import lax
from jax.experimental import pallas as pl
from jax.experimental.pallas import tpu as pltpu
from jax.experimental.pallas import tpu_sc as plsc

_ALPHA = 0.2
_L = 16
_NC = 2
_NS = 16
_B = 80
_D = 128
_DR = 80
_NP = 10368
_DR0 = 10240


def _matmul(x, w):
    n, d_in = x.shape
    d_out = w.shape[1]
    rows = n // 10

    def body(x_ref, w_ref, o_ref):
        o_ref[...] = jnp.dot(x_ref[...], w_ref[...],
                             preferred_element_type=jnp.float32)

    return pl.pallas_call(
        body,
        grid=(10,),
        in_specs=[
            pl.BlockSpec((rows, d_in), lambda i: (i, 0)),
            pl.BlockSpec((d_in, d_out), lambda i: (0, 0)),
        ],
        out_specs=pl.BlockSpec((rows, d_out), lambda i: (i, 0)),
        out_shape=jax.ShapeDtypeStruct((n, d_out), jnp.float32),
    )(x, w)


def _permute(v, idx):
    return lax.gather(
        v, idx[:, None],
        lax.GatherDimensionNumbers(offset_dims=(), collapsed_slice_dims=(0,),
                                   start_index_map=(0,)),
        (1,), mode=lax.GatherScatterMode.PROMISE_IN_BOUNDS)


def _sc_attention(h, src3, dst3):
    n = h.shape[0]
    nw, nblk, _ = src3.shape
    nck = _D // _L
    zpt = _NP // _NS

    mesh = plsc.VectorSubcoreMesh(core_axis_name="c", subcore_axis_name="s")

    @functools.partial(
        pl.kernel,
        out_type=jax.ShapeDtypeStruct((_NC * _NP, _D), jnp.float32),
        mesh=mesh,
        scratch_types=[
            pltpu.VMEM((_B,), jnp.int32),
            pltpu.VMEM((_B,), jnp.int32),
            pltpu.VMEM((_B,), jnp.int32),
            pltpu.VMEM((_B,), jnp.int32),
            pltpu.VMEM((_B,), jnp.int32),
            pltpu.VMEM((_B, _D), jnp.float32),
            pltpu.VMEM((_B, _D), jnp.float32),
            pltpu.VMEM((_B, _D), jnp.float32),
            pltpu.VMEM((_DR * _D,), jnp.float32),
            pltpu.VMEM((_B,), jnp.float32),
            pltpu.VMEM((_DR,), jnp.int32),
            pltpu.VMEM_SHARED((_NP, _D), jnp.float32),
            pltpu.SemaphoreType.DMA,
            pltpu.SemaphoreType.DMA,
            pltpu.SemaphoreType.DMA,
            pltpu.SemaphoreType.DMA,
        ],
    )
    def k(h_hbm, src_hbm, dst_hbm, acc_out,
          sid0, sid1, did0, did1, sc_idx, rows_s, rows_d0, rows_d1, den_f,
          sq_buf, den_idx, acc_sh, sem_gs, sem_gd, sem_sc, sem_id):
        cid = lax.axis_index("c")
        sid = lax.axis_index("s")
        wid = sid * _NC + cid
        lane = lax.iota(jnp.int32, _L)
        zf = jnp.zeros((_L,), jnp.float32)

        def zrow(i, carry):
            for kq in range(nck):
                rows_s[i, pl.ds(kq * _L, _L)] = zf
            return carry
        lax.fori_loop(0, _B, zrow, 0)

        def zden(i, carry):
            den_f[pl.ds(i * _L, _L)] = zf
            return carry
        lax.fori_loop(0, _DR * _D // _L, zden, 0)

        for kq in range(_DR // _L):
            den_idx[pl.ds(kq * _L, _L)] = _DR0 + kq * _L + lane

        zb = sid * zpt
        for part in range(zpt // _B):
            pltpu.sync_copy(rows_s, acc_sh.at[pl.ds(zb + part * _B, _B)])
        pltpu.sync_copy(rows_s.at[pl.ds(0, zpt % _B)],
                        acc_sh.at[pl.ds(zb + zpt - zpt % _B, zpt % _B)])
        plsc.subcore_barrier()

        def sqdist_block(sidb, cur):
            def groupA(g, gcarry):
                base = g * _L
                sv16 = sidb[pl.ds(base, _L)]
                sc_idx[pl.ds(base, _L)] = sv16
                accs = [zf] * _L
                for kq in range(nck):
                    for j in range(_L):
                        ei = base + j
                        sv = rows_s[ei, pl.ds(kq * _L, _L)]
                        t = cur[ei, pl.ds(kq * _L, _L)]
                        df = sv - t
                        accs[j] = accs[j] + df * df
                bit = 1
                while len(accs) > 1:
                    cond = (lane & bit) != 0
                    folded = []
                    for i in range(0, len(accs), 2):
                        a, b2 = accs[i], accs[i + 1]
                        x = jnp.where(cond, b2, a)
                        z = jnp.where(cond, a, b2)
                        folded.append(x + _permute(z, lane ^ bit))
                    accs = folded
                    bit <<= 1
                sq_buf[pl.ds(base, _L)] = accs[0]
                return gcarry
            lax.fori_loop(0, _B // _L, groupA, 0)

        def scale_block(sidb, cur):
            def groupC(g, gcarry):
                base = g * _L
                sv16 = sidb[pl.ds(base, _L)]
                ee16 = jnp.exp(_ALPHA * sq_buf[pl.ds(base, _L)])
                ebs = [_permute(ee16, jnp.full((_L,), j, jnp.int32))
                       for j in range(_L)]
                for j in range(_L):
                    srcs = sv16[j]
                    p16 = (srcs >> 4) << 4
                    sel = lane == (srcs & 15)
                    den_f[pl.ds(p16, _L)] = (den_f[pl.ds(p16, _L)]
                                             + jnp.where(sel, ebs[j], 0.0))
                for kq in range(nck):
                    for j in range(_L):
                        ei = base + j
                        cur[ei, pl.ds(kq * _L, _L)] = (
                            ebs[j] * cur[ei, pl.ds(kq * _L, _L)])
                return gcarry
            lax.fori_loop(0, _B // _L, groupC, 0)

        def body(b, cur, nxt, sidb, didb, sidn, didn, first):
            pltpu.make_async_copy(h_hbm.at[sidb], rows_s, sem_gs).wait()
            pltpu.make_async_copy(h_hbm.at[didb], cur, sem_gd).wait()

            if not first:
                pltpu.make_async_copy(nxt, acc_sh.at[sc_idx], sem_sc).wait()

            @pl.when(b < nblk - 1)
            def _issue_next_dst_gather():
                pltpu.make_async_copy(src_hbm.at[wid, b + 1], sidn,
                                      sem_id).wait()
                pltpu.make_async_copy(dst_hbm.at[wid, b + 1], didn,
                                      sem_id).wait()
                pltpu.async_copy(h_hbm.at[didn], nxt, sem_gd)

            sqdist_block(sidb, cur)

            @pl.when(b < nblk - 1)
            def _issue_next_src_gather():
                pltpu.async_copy(h_hbm.at[sidn], rows_s, sem_gs)

            scale_block(sidb, cur)
            pltpu.async_copy(cur, acc_sh.at[sc_idx], sem_sc, add=True)

            @pl.when(b < nblk - 2)
            def _prefetch_ids():
                pltpu.async_copy(src_hbm.at[wid, b + 2], sidb, sem_id)
                pltpu.async_copy(dst_hbm.at[wid, b + 2], didb, sem_id)

        pltpu.sync_copy(src_hbm.at[wid, 0], sid0)
        pltpu.sync_copy(dst_hbm.at[wid, 0], did0)
        pltpu.async_copy(h_hbm.at[sid0], rows_s, sem_gs)
        pltpu.async_copy(h_hbm.at[did0], rows_d0, sem_gd)
        pltpu.async_copy(src_hbm.at[wid, 1], sid1, sem_id)
        pltpu.async_copy(dst_hbm.at[wid, 1], did1, sem_id)
        body(jnp.int32(0), rows_d0, rows_d1, sid0, did0, sid1, did1, True)

        def pair(kk, carry):
            b1 = 2 * kk + 1
            body(b1, rows_d1, rows_d0, sid1, did1, sid0, did0, False)
            body(b1 + 1, rows_d0, rows_d1, sid0, did0, sid1, did1, False)
            return carry
        lax.fori_loop(0, (nblk - 1) // 2, pair, 0)

        pltpu.make_async_copy(rows_d0, acc_sh.at[sc_idx], sem_sc).wait()

        def cvt(i, carry):
            for kq in range(nck):
                rows_s[i, pl.ds(kq * _L, _L)] = den_f[pl.ds(i * _D + kq * _L,
                                                            _L)]
            return carry
        lax.fori_loop(0, _DR, cvt, 0)
        pltpu.sync_copy(rows_s, acc_sh.at[den_idx], add=True)

        plsc.subcore_barrier()
        pltpu.sync_copy(acc_sh.at[pl.ds(zb, zpt)],
                        acc_out.at[pl.ds(cid * _NP + zb, zpt)])

    return k(h, src3, dst3)


def _combine(acc, n):
    def body(acc_ref, o_ref):
        s = acc_ref[0] + acc_ref[1]
        num = s[:n]
        dsum = s[_DR0:_DR0 + _DR]
        row = lax.broadcasted_iota(jnp.int32, (n, _DR), 0)
        blkid = lax.broadcasted_iota(jnp.int32, (n, _DR), 1)
        a = ((row >> 7) == blkid).astype(jnp.float32)
        dblk = jnp.dot(a, dsum, preferred_element_type=jnp.float32)
        r2 = lax.broadcasted_iota(jnp.int32, (n, _D), 0)
        c2 = lax.broadcasted_iota(jnp.int32, (n, _D), 1)
        cmask = ((r2 & 127) == c2).astype(jnp.float32)
        dcol = jnp.sum(dblk * cmask, axis=1, keepdims=True)
        o_ref[...] = jnp.maximum(num / dcol, 0.0)

    return pl.pallas_call(
        body,
        out_shape=jax.ShapeDtypeStruct((n, _D), jnp.float32),
    )(acc)


def kernel(x, edge_index, W):
    n = x.shape[0]
    e = edge_index.shape[1]
    nw = _NC * _NS
    epw = e // nw
    h = _matmul(x, W)
    src3 = edge_index[0].reshape(nw, epw // _B, _B)
    dst3 = edge_index[1].reshape(nw, epw // _B, _B)
    acc = _sc_attention(h, src3, dst3)
    return _combine(acc.reshape(_NC, _NP, _D), n)
